# baseline (speedup 1.0000x reference)
import functools

import jax
import jax.numpy as jnp
import numpy as np
from jax import lax
from jax.experimental import pallas as pl
from jax.experimental.pallas import tpu as pltpu
from jax.experimental.pallas import tpu_sc as plsc

_MU = 255.0
_EMBED_NUM = 256
_HIDDEN = 64

_ROWS = 16384
_COLS = 50
_B = _ROWS * _COLS
_IDX_COLS = 128
_IDX_ROWS = _B // _IDX_COLS

_NC = 2
_NS = 16
_NW = _NC * _NS
_ROWS_PER_W = _ROWS // _NW
_CR = 4
_CHUNK = _CR * _COLS
_N_CHUNKS = _ROWS_PER_W // _CR
_BATCH_CHUNKS = 16
_BATCH = _BATCH_CHUNKS * _CHUNK
_L = 16


def _mulaw_index_body(x_ref, o_ref):
    v = x_ref[...]
    s = jnp.sign(v)
    x = s * jnp.log(1.0 + _MU * jnp.abs(v)) / np.log(1.0 + _MU)
    idx = jnp.floor((x + 1.0) * (_EMBED_NUM // 2)).astype(jnp.int32)
    lo = (idx >= 0).astype(jnp.int32)
    mid = (idx < _EMBED_NUM).astype(jnp.int32)
    hi = (idx >= _EMBED_NUM).astype(jnp.int32)
    o_ref[...] = lo * mid * idx + hi * (_EMBED_NUM - 1)


_mulaw_index = pl.pallas_call(
    _mulaw_index_body,
    out_shape=jax.ShapeDtypeStruct((_IDX_ROWS, _IDX_COLS), jnp.int32),
)


def _gather_body(table_hbm, idx_hbm, out_hbm,
                 table_v, idx_v, rows0, rows1, sem0, sem1):
    wid = lax.axis_index("s") * _NC + lax.axis_index("c")
    row0 = wid * _ROWS_PER_W
    base = wid * (_ROWS_PER_W * _COLS)

    pltpu.sync_copy(table_hbm, table_v)

    def do_chunk(i, b, rows_v, sem):
        ci = 2 * i + b
        cb = lax.rem(ci, _BATCH_CHUNKS) * _CHUNK

        @pl.when(i > 0)
        def _drain():
            pltpu.make_async_copy(
                rows_v, out_hbm.at[pl.ds(0, _CR)], sem).wait()

        for w in range(-(-_CHUNK // _L)):
            win = pl.multiple_of(cb + w * _L, 8)
            iv = idx_v[pl.ds(win, _L)] * _HIDDEN
            for j in range(_L):
                s = w * _L + j
                if s >= _CHUNK:
                    break
                r, c = divmod(s, _COLS)
                src = pl.multiple_of(iv[j], _HIDDEN)
                for k in range(0, _HIDDEN, _L):
                    rows_v[r, c, pl.ds(k, _L)] = table_v[pl.ds(src + k, _L)]

        pltpu.async_copy(
            rows_v, out_hbm.at[pl.ds(row0 + ci * _CR, _CR)], sem)

    def pair(i, carry):
        @pl.when(lax.rem(i, _BATCH_CHUNKS // 2) == 0)
        def _stage_idx():
            bi = lax.div(i, _BATCH_CHUNKS // 2)
            pltpu.sync_copy(
                idx_hbm.at[pl.ds(base + bi * _BATCH, _BATCH + _L)], idx_v)

        do_chunk(i, 0, rows0, sem0)
        do_chunk(i, 1, rows1, sem1)
        return carry

    lax.fori_loop(0, _N_CHUNKS // 2, pair, 0)
    pltpu.make_async_copy(rows0, out_hbm.at[pl.ds(0, _CR)], sem0).wait()
    pltpu.make_async_copy(rows1, out_hbm.at[pl.ds(0, _CR)], sem1).wait()


_gather = functools.partial(
    pl.kernel,
    mesh=plsc.VectorSubcoreMesh(core_axis_name="c", subcore_axis_name="s"),
    out_type=jax.ShapeDtypeStruct((_ROWS, _COLS, _HIDDEN), jnp.float32),
    scratch_types=[
        pltpu.VMEM((_EMBED_NUM * _HIDDEN,), jnp.float32),
        pltpu.VMEM((_BATCH + _L,), jnp.int32),
        pltpu.VMEM((_CR, _COLS, _HIDDEN), jnp.float32),
        pltpu.VMEM((_CR, _COLS, _HIDDEN), jnp.float32),
        pltpu.SemaphoreType.DMA,
        pltpu.SemaphoreType.DMA,
    ],
    compiler_params=pltpu.CompilerParams(use_tc_tiling_on_sc=False,
                                         needs_layout_passes=False),
)(_gather_body)


def kernel(index, W):
    idx = _mulaw_index(index.reshape(_IDX_ROWS, _IDX_COLS))
    idx = jnp.concatenate([idx.reshape(-1), jnp.zeros((_L,), jnp.int32)])
    return _gather(W.reshape(-1), idx)
